# baseline (speedup 1.0000x reference)
import functools

import jax
import jax.numpy as jnp
from jax import lax
from jax.experimental import pallas as pl
from jax.experimental.pallas import tpu as pltpu
from jax.experimental.pallas import tpu_sc as plsc

N = 50000
E = 800000
H_DIM = 16
LAT = 32
HID = 64
MLP_HID = 128

EPAD = 819200
NIDXROWS = EPAD // 128
NC = 2
NS = 16
NW = NC * NS
ROWS_PER_W = NIDXROWS // NW
NACC = 50048

BN = 2000
BE = 3200
E4 = E // 4
EPAD4 = EPAD // 4
BE4 = 2000

_mesh = plsc.VectorSubcoreMesh(core_axis_name="c", subcore_axis_name="s")
_sc_params = pltpu.CompilerParams(use_tc_tiling_on_sc=False)


@functools.lru_cache(maxsize=None)
def _gather2_kernel(D=LAT):
    G = 2 if D >= LAT else 4
    CH = G * 128
    NG = ROWS_PER_W // G
    trows = N // NS

    @functools.partial(
        pl.kernel,
        mesh=_mesh,
        compiler_params=_sc_params,
        out_type=[jax.ShapeDtypeStruct((EPAD, D), jnp.float32),
                  jax.ShapeDtypeStruct((EPAD, D), jnp.float32)],
        scratch_types=[pltpu.VMEM((G, 128), jnp.int32),
                       pltpu.VMEM((G, 128), jnp.int32),
                       pltpu.VMEM((CH, D), jnp.float32),
                       pltpu.VMEM((CH, D), jnp.float32),
                       pltpu.VMEM_SHARED((NACC, D), jnp.float32),
                       pltpu.SemaphoreType.DMA,
                       pltpu.SemaphoreType.DMA],
    )
    def k(table, src2d, dst2d, o_src, o_dst, idx0, idx1, rows0, rows1,
          tspm, sem0, sem1):
        s = lax.axis_index("s")
        wid = s * NC + lax.axis_index("c")
        row0 = wid * ROWS_PER_W
        e0 = row0 * 128
        pltpu.sync_copy(table.at[pl.ds(s * trows, trows)],
                        tspm.at[pl.ds(s * trows, trows)])
        plsc.subcore_barrier()

        for li, out in ((0, o_src), (1, o_dst)):
            ind = src2d if li == 0 else dst2d

            def stage(g, idx):
                pltpu.sync_copy(ind.at[pl.ds(row0 + g * G, G)], idx)

            def issue(idx, rows, sem):
                for j in range(G):
                    pltpu.async_copy(tspm.at[idx.at[j]],
                                     rows.at[pl.ds(j * 128, 128)], sem)

            def drain(rows, sem):
                pltpu.make_async_copy(table.at[pl.ds(0, CH)], rows, sem).wait()

            def wb(g, rows):
                pltpu.sync_copy(rows, out.at[pl.ds(e0 + g * CH, CH)])

            stage(0, idx0)
            issue(idx0, rows0, sem0)
            npairs = (NG - 2) // 2

            def body(t, carry):
                g1 = 2 * t + 1
                stage(g1, idx1)
                issue(idx1, rows1, sem1)
                drain(rows0, sem0)
                wb(g1 - 1, rows0)
                stage(g1 + 1, idx0)
                issue(idx0, rows0, sem0)
                drain(rows1, sem1)
                wb(g1, rows1)
                return carry

            lax.fori_loop(0, npairs, body, 0)
            stage(NG - 1, idx1)
            issue(idx1, rows1, sem1)
            drain(rows0, sem0)
            wb(NG - 2, rows0)
            drain(rows1, sem1)
            wb(NG - 1, rows1)

    return k


SC_G = 2
SC_CH = SC_G * 128
SC_NG = ROWS_PER_W // SC_G


@functools.lru_cache(maxsize=None)
def _scatter_kernel():
    zrows = NACC // NS

    @functools.partial(
        pl.kernel,
        mesh=_mesh,
        compiler_params=_sc_params,
        out_type=jax.ShapeDtypeStruct((NC, NACC, LAT), jnp.float32),
        scratch_types=[pltpu.VMEM((SC_G, 128), jnp.int32),
                       pltpu.VMEM((SC_G, 128), jnp.int32),
                       pltpu.VMEM((SC_CH, LAT), jnp.float32),
                       pltpu.VMEM((SC_CH, LAT), jnp.float32),
                       pltpu.VMEM_SHARED((NACC, LAT), jnp.float32),
                       pltpu.SemaphoreType.DMA,
                       pltpu.SemaphoreType.DMA],
    )
    def k(m, dst2d, zeros_hbm, partial, idx0, idx1, rows0, rows1, acc,
          sem0, sem1):
        c = lax.axis_index("c")
        s = lax.axis_index("s")
        wid = s * NC + c
        row0 = wid * ROWS_PER_W
        e0 = row0 * 128

        pltpu.sync_copy(zeros_hbm.at[pl.ds(s * zrows, zrows)],
                        acc.at[pl.ds(s * zrows, zrows)])
        plsc.subcore_barrier()

        def stage(g, idx, rows):
            pltpu.sync_copy(dst2d.at[pl.ds(row0 + g * SC_G, SC_G)], idx)
            pltpu.sync_copy(m.at[pl.ds(e0 + g * SC_CH, SC_CH)], rows)

        def issue(idx, rows, sem):
            for j in range(SC_G):
                pltpu.async_copy(rows.at[pl.ds(j * 128, 128)],
                                 acc.at[idx.at[j]], sem, add=True)

        def drain(rows, sem):
            pltpu.make_async_copy(rows, acc.at[pl.ds(0, SC_CH)], sem).wait()

        stage(0, idx0, rows0)
        issue(idx0, rows0, sem0)
        npairs = (SC_NG - 2) // 2

        def body(t, carry):
            g1 = 2 * t + 1

            @pl.when(t > 0)
            def _():
                drain(rows1, sem1)

            stage(g1, idx1, rows1)
            issue(idx1, rows1, sem1)
            drain(rows0, sem0)
            stage(g1 + 1, idx0, rows0)
            issue(idx0, rows0, sem0)
            return carry

        lax.fori_loop(0, npairs, body, 0)
        drain(rows1, sem1)
        stage(SC_NG - 1, idx1, rows1)
        issue(idx1, rows1, sem1)
        drain(rows0, sem0)
        drain(rows1, sem1)

        plsc.subcore_barrier()
        pltpu.sync_copy(acc.at[pl.ds(s * zrows, zrows)],
                        partial.at[c, pl.ds(s * zrows, zrows)])

    return k


def _full(shape):
    return pl.BlockSpec(shape, lambda i: tuple(0 for _ in shape))


def _silu(x):
    return x * jax.nn.sigmoid(x)


def _dot(a, b):
    return jnp.dot(a, b, preferred_element_type=jnp.float32)


def _dotx(a, b):
    return jnp.dot(a, b, preferred_element_type=jnp.float32,
                   precision=jax.lax.Precision.HIGHEST)


def _pad_idx(v, fill):
    return jnp.pad(v, (0, EPAD - E),
                   constant_values=fill).reshape(NIDXROWS, 128)


def _mlp1(h, W, b):
    din, dout = W.shape

    def body(h_ref, w_ref, b_ref, o_ref):
        o_ref[...] = _silu(_dot(h_ref[...], w_ref[...]) + b_ref[...])

    return pl.pallas_call(
        body,
        grid=(N // BN,),
        in_specs=[pl.BlockSpec((BN, din), lambda i: (i, 0)),
                  _full((din, dout)), _full((1, dout))],
        out_specs=pl.BlockSpec((BN, dout), lambda i: (i, 0)),
        out_shape=jax.ShapeDtypeStruct((N, dout), jnp.float32),
    )(h, W, b.reshape(1, dout))


def _proj(h, W, b):
    din, dout = W.shape

    def body(h_ref, w_ref, b_ref, o_ref):
        o_ref[...] = _dot(h_ref[...], w_ref[...]) + b_ref[...]

    return pl.pallas_call(
        body,
        grid=(N // BN,),
        in_specs=[pl.BlockSpec((BN, din), lambda i: (i, 0)),
                  _full((din, dout)), _full((1, dout))],
        out_specs=pl.BlockSpec((BN, dout), lambda i: (i, 0)),
        out_shape=jax.ShapeDtypeStruct((N, dout), jnp.float32),
    )(h, W, b.reshape(1, dout))


def _aux_blockdiag_ones():
    out = jnp.zeros((128, 4), jnp.float32)
    for j in range(4):
        out = out.at[32 * j:32 * (j + 1), j].set(1.0)
    return out


def _aux_perms():
    s1 = jnp.zeros((4, 32), jnp.float32)
    s2 = jnp.zeros((16, 32), jnp.float32)
    for j in range(4):
        s1 = s1.at[j, 8 * j].set(1.0)
        for kk in range(4):
            s2 = s2.at[4 * j + kk, 8 * j + 1 + kk].set(1.0)
    return s1, s2


def _d2aux(pairs, ea_p):
    onesb = _aux_blockdiag_ones()
    s1, s2 = _aux_perms()
    with_ea = ea_p is not None
    npairs = len(pairs)

    def body(*refs):
        o_ref = refs[-1]
        d2m = None
        for i in range(npairs):
            r = refs[2 * i][...] - refs[2 * i + 1][...]
            sq = r * r
            t = jnp.concatenate(
                [jnp.sum(sq[:, 32 * kk:32 * kk + 32], axis=1, keepdims=True)
                 for kk in range(4)], axis=1)
            d2m = t if d2m is None else d2m + t
        aux32 = _dotx(d2m, refs[-3][...])
        if with_ea:
            aux32 = aux32 + _dotx(refs[2 * npairs][...], refs[-2][...])
        o_ref[...] = jnp.concatenate(
            [aux32, jnp.zeros((BE4, 96), jnp.float32)], axis=1)

    blk = pl.BlockSpec((BE4, 128), lambda i: (i, 0))
    args = []
    in_specs = []
    for xs_p, xd_p in pairs:
        args += [xs_p, xd_p]
        in_specs += [blk, blk]
    if with_ea:
        args.append(ea_p)
        in_specs.append(pl.BlockSpec((BE4, 16), lambda i: (i, 0)))
    args += [onesb, s1, s2]
    in_specs += [_full((128, 4)), _full((4, 32)), _full((16, 32))]

    return pl.pallas_call(
        body,
        grid=(E4 // BE4,),
        in_specs=in_specs,
        out_specs=blk,
        out_shape=jax.ShapeDtypeStruct((E4, 128), jnp.float32),
    )(*args)


def _edge_mlp(hd_p, hs_p, aux_p, lp):
    W1 = jnp.pad(lp["e1"]["W"], ((0, 3), (0, 0)))
    b1 = lp["e1"]["b"].reshape(1, HID)
    W2 = lp["e2"]["W"]
    b2 = lp["e2"]["b"].reshape(1, LAT)

    def body(hd_ref, hs_ref, aux_ref, w1_r, b1_r, w2_r, b2_r, o_ref):
        hdv = hd_ref[...]
        hsv = hs_ref[...]
        auxv = aux_ref[...]
        outs = []
        for kk in range(4):
            m_in = jnp.concatenate(
                [hdv[:, 32 * kk:32 * kk + 32], hsv[:, 32 * kk:32 * kk + 32],
                 auxv[:, 8 * kk:8 * kk + 5],
                 jnp.zeros((BE4, 3), jnp.float32)], axis=1)
            u = _silu(_dot(m_in, w1_r[...]) + b1_r[...])
            outs.append(_silu(_dot(u, w2_r[...]) + b2_r[...]))
        o_ref[...] = jnp.concatenate(outs, axis=1)

    blk = pl.BlockSpec((BE4, 128), lambda i: (i, 0))
    return pl.pallas_call(
        body,
        grid=(E4 // BE4,),
        in_specs=[blk, blk, blk,
                  _full((2 * LAT + 8, HID)), _full((1, HID)),
                  _full((HID, LAT)), _full((1, LAT))],
        out_specs=blk,
        out_shape=jax.ShapeDtypeStruct((EPAD4, 128), jnp.float32),
    )(hd_p, hs_p, aux_p, W1, b1, W2, b2)


def _node_update(hh, part, lp):
    Wh1 = lp["h1"]["W"]
    bh1 = lp["h1"]["b"].reshape(1, HID)
    Wh2 = lp["h2"]["W"]
    bh2 = lp["h2"]["b"].reshape(1, LAT)
    p0 = part[0]
    p1 = part[1]

    def body(hh_ref, p0_ref, p1_ref, w1_r, b1_r, w2_r, b2_r, o_ref):
        agg = p0_ref[...] + p1_ref[...]
        cat = jnp.concatenate([hh_ref[...], agg], axis=1)
        u = _silu(_dot(cat, w1_r[...]) + b1_r[...])
        o_ref[...] = hh_ref[...] + _dot(u, w2_r[...]) + b2_r[...]

    return pl.pallas_call(
        body,
        grid=(N // BN,),
        in_specs=[pl.BlockSpec((BN, LAT), lambda i: (i, 0)),
                  pl.BlockSpec((BN, LAT), lambda i: (i, 0)),
                  pl.BlockSpec((BN, LAT), lambda i: (i, 0)),
                  _full((2 * LAT, HID)), _full((1, HID)),
                  _full((HID, LAT)), _full((1, LAT))],
        out_specs=pl.BlockSpec((BN, LAT), lambda i: (i, 0)),
        out_shape=jax.ShapeDtypeStruct((N, LAT), jnp.float32),
    )(hh, p0, p1, Wh1, bh1, Wh2, bh2)


def _softplus(x):
    return jnp.maximum(x, 0.0) + jnp.log1p(jnp.exp(-jnp.abs(x)))


def _vae_prior(zgp, p):
    L1, b1 = p["l1"]["W"], p["l1"]["b"].reshape(1, MLP_HID)
    L2, b2 = p["l2"]["W"], p["l2"]["b"].reshape(1, 2 * LAT)

    def body(z_ref, l1_r, b1_r, l2_r, b2_r, loc_ref, scale_ref):
        hdn = _silu(_dot(z_ref[...], l1_r[...]) + b1_r[...])
        o = _dot(hdn, l2_r[...]) + b2_r[...]
        loc_ref[...] = o[:, 0:LAT]
        scale_ref[...] = _softplus(o[:, LAT:2 * LAT]) + 1e-4

    return pl.pallas_call(
        body,
        grid=(N // BN,),
        in_specs=[pl.BlockSpec((BN, LAT), lambda i: (i, 0)),
                  _full((LAT, MLP_HID)), _full((1, MLP_HID)),
                  _full((MLP_HID, 2 * LAT)), _full((1, 2 * LAT))],
        out_specs=[pl.BlockSpec((BN, LAT), lambda i: (i, 0)),
                   pl.BlockSpec((BN, LAT), lambda i: (i, 0))],
        out_shape=[jax.ShapeDtypeStruct((N, LAT), jnp.float32),
                   jax.ShapeDtypeStruct((N, LAT), jnp.float32)],
    )(zgp, L1, b1, L2, b2)


def _vae_inf(zg, zgp, eps, p):
    L1 = p["l1"]["W"]
    b1 = p["l1"]["b"].reshape(1, MLP_HID)
    L2, b2 = p["l2"]["W"], p["l2"]["b"].reshape(1, 2 * LAT)

    def body(zg_ref, zgp_ref, eps_ref, l1_r, b1_r, l2_r, b2_r,
             loc_ref, scale_ref, z_ref):
        cat = jnp.concatenate([zg_ref[...], zgp_ref[...]], axis=1)
        hdn = _silu(_dot(cat, l1_r[...]) + b1_r[...])
        o = _dot(hdn, l2_r[...]) + b2_r[...]
        loc = o[:, 0:LAT]
        scale = _softplus(o[:, LAT:2 * LAT]) + 1e-4
        loc_ref[...] = loc
        scale_ref[...] = scale
        z_ref[...] = loc + scale * eps_ref[...]

    return pl.pallas_call(
        body,
        grid=(N // BN,),
        in_specs=[pl.BlockSpec((BN, LAT), lambda i: (i, 0)),
                  pl.BlockSpec((BN, LAT), lambda i: (i, 0)),
                  pl.BlockSpec((BN, LAT), lambda i: (i, 0)),
                  _full((2 * LAT, MLP_HID)), _full((1, MLP_HID)),
                  _full((MLP_HID, 2 * LAT)), _full((1, 2 * LAT))],
        out_specs=[pl.BlockSpec((BN, LAT), lambda i: (i, 0)),
                   pl.BlockSpec((BN, LAT), lambda i: (i, 0)),
                   pl.BlockSpec((BN, LAT), lambda i: (i, 0))],
        out_shape=[jax.ShapeDtypeStruct((N, LAT), jnp.float32),
                   jax.ShapeDtypeStruct((N, LAT), jnp.float32),
                   jax.ShapeDtypeStruct((N, LAT), jnp.float32)],
    )(zg, zgp, eps, L1, b1, L2, b2)


def _gnn_pass(params, tables, h, ea_p, src2d, dst2d, zeros_acc):
    hh = _mlp1(h, params["embed"]["W"], params["embed"]["b"])
    pairs = []
    for t in tables:
        xs, xd = _gather2_kernel()(t, src2d, dst2d)
        pairs.append((xs.reshape(EPAD4, 128), xd.reshape(EPAD4, 128)))
    aux = _d2aux(pairs, ea_p)
    for lp in params["layers"]:
        hs, hd = _gather2_kernel()(hh, src2d, dst2d)
        m_p = _edge_mlp(hd.reshape(EPAD4, 128), hs.reshape(EPAD4, 128),
                        aux, lp)
        part = _scatter_kernel()(m_p.reshape(EPAD, LAT), dst2d, zeros_acc)
        hh = _node_update(hh, part, lp)
    return _proj(hh, params["out"]["W"], params["out"]["b"])


def kernel(x, h, edge_attr, edge_attr_partial, edge_index, partial_goal_mask,
           enc_goal_params, enc_partial_params, dec_params, inf_params,
           prior_params):
    src = edge_index[0]
    dst = edge_index[1]
    src2d = _pad_idx(src, 0)
    dst2d = _pad_idx(dst, N)
    zeros_acc = jnp.zeros((NACC, LAT), jnp.float32)

    x_pad = jnp.pad(x, ((0, 0), (0, LAT - 3)))
    xp_pad = partial_goal_mask[:, None] * x_pad
    ea_p = edge_attr.reshape(E4, 16)
    eap_p = edge_attr_partial.reshape(E4, 16)

    z_goal = _gnn_pass(enc_goal_params, (x_pad,), h, ea_p,
                       src2d, dst2d, zeros_acc)
    z_goal_partial = _gnn_pass(enc_partial_params, (xp_pad,), h,
                               eap_p, src2d, dst2d, zeros_acc)

    p_loc, p_scale = _vae_prior(z_goal_partial, prior_params)
    eps = jax.random.normal(jax.random.key(42), (N, LAT), jnp.float32)
    q_loc, q_scale, z = _vae_inf(z_goal, z_goal_partial, eps, inf_params)

    mu_x_sample = _gnn_pass(dec_params, (z, z_goal_partial), h, None,
                            src2d, dst2d, zeros_acc)
    return (mu_x_sample, q_loc, q_scale, p_loc, p_scale)
